# baseline (speedup 1.0000x reference)
import jax
import jax.numpy as jnp
from jax import lax
from jax.experimental import pallas as pl
from jax.experimental.pallas import tpu as pltpu

N = 4096
BLK = 512
NBLK = N // BLK
NUM_BINS = 3
K0 = N // NUM_BINS
K1 = 2 * (N // NUM_BINS)
MAX_FINITE_BITS = 0x7F7FFFFF


def _fused_kernel(s2d_ref, s_row_ref, w_hbm, out_ref,
                  qmin_ref, qmax_ref, b0, b1, sem0, sem1):
    j = pl.program_id(0)
    bufs = (b0, b1)
    sems = (sem0, sem1)

    @pl.when(j == 0)
    def _start0():
        pltpu.make_async_copy(
            w_hbm.at[:, pl.ds(0, BLK)], b0, sem0).start()

    @pl.when(j == 0)
    def _bin():
        s2d = s2d_ref[:]

        def cnt_le(vbits):
            v = lax.bitcast_convert_type(vbits, jnp.float32)
            le = jnp.where(s2d <= v, 1.0, 0.0)
            return jnp.sum(le, axis=(0, 1), keepdims=True)

        def narrow(lo, hi, k_rank):
            ln = hi - lo
            step = jnp.maximum(jnp.right_shift(ln, 4), 1)
            idx = jnp.full((1, 1), 15, jnp.int32)
            for k in range(1, 16):
                b = cnt_le(lo + step * k) >= k_rank + 1
                idx = idx - jnp.where(b, 1, 0)
            nlo = lo + step * idx
            nhi = jnp.where(idx == 15, hi,
                            jnp.minimum(lo + step * (idx + 1), hi))
            return nlo, nhi

        def body(_, carry):
            lo0, hi0, lo1, hi1 = carry
            lo0, hi0 = narrow(lo0, hi0, K0)
            lo1, hi1 = narrow(lo1, hi1, K1)
            return lo0, hi0, lo1, hi1

        lo = jnp.full((1, 1), -1, jnp.int32)
        hi = jnp.full((1, 1), MAX_FINITE_BITS, jnp.int32)
        _, hi0, _, hi1 = lax.fori_loop(0, 10, body, (lo, hi, lo, hi))
        t0 = lax.bitcast_convert_type(hi0, jnp.float32)
        t1 = lax.bitcast_convert_type(hi1, jnp.float32)
        s_row = s_row_ref[:]
        half = jnp.where(s_row <= t0, 2.0, jnp.where(s_row <= t1, 8.0, 32.0))
        qmin_ref[:] = -half
        qmax_ref[:] = half - 1.0

    @pl.when(j + 1 < NBLK)
    def _prefetch():
        for par in (0, 1):
            @pl.when((j + 1) % 2 == par)
            def _(par=par):
                pltpu.make_async_copy(
                    w_hbm.at[:, pl.ds((j + 1) * BLK, BLK)],
                    bufs[par], sems[par]).start()

    q_min = qmin_ref[:, pl.ds(j * BLK, BLK)]
    q_max = qmax_ref[:, pl.ds(j * BLK, BLK)]
    for par in (0, 1):
        @pl.when(j % 2 == par)
        def _(par=par):
            pltpu.make_async_copy(
                w_hbm.at[:, pl.ds(j * BLK, BLK)],
                bufs[par], sems[par]).wait()
            w = bufs[par][:]
            mn = jnp.min(w, axis=0, keepdims=True)
            mx = jnp.max(w, axis=0, keepdims=True)
            scale = (mx - mn) / (q_max - q_min)
            scale = jnp.where(jnp.abs(scale) < 1e-6, jnp.float32(1e-6), scale)
            zp = jnp.clip(jnp.round(q_min - mn / scale), q_min, q_max)
            q = jnp.clip(jnp.round(w / scale), -128.0 - zp, 127.0 - zp)
            out_ref[:] = q * scale


def kernel(weight, scores):
    s_row = scores.reshape(1, N)
    s2d = scores.reshape(8, N // 8)
    out = pl.pallas_call(
        _fused_kernel,
        grid=(NBLK,),
        in_specs=[
            pl.BlockSpec((8, N // 8), lambda j: (0, 0)),
            pl.BlockSpec((1, N), lambda j: (0, 0)),
            pl.BlockSpec(memory_space=pl.ANY),
        ],
        out_specs=pl.BlockSpec((N, BLK), lambda j: (0, j)),
        out_shape=jax.ShapeDtypeStruct((N, N), jnp.float32),
        scratch_shapes=[
            pltpu.VMEM((1, N), jnp.float32),
            pltpu.VMEM((1, N), jnp.float32),
            pltpu.VMEM((N, BLK), jnp.float32),
            pltpu.VMEM((N, BLK), jnp.float32),
            pltpu.SemaphoreType.DMA,
            pltpu.SemaphoreType.DMA,
        ],
        compiler_params=pltpu.CompilerParams(
            dimension_semantics=("arbitrary",),
        ),
    )(s2d, s_row, weight)
    return out


# speedup vs baseline: 1.0589x; 1.0589x over previous
import jax
import jax.numpy as jnp
from jax import lax
from jax.experimental import pallas as pl
from jax.experimental.pallas import tpu as pltpu

N = 4096
BLK = 512
NBLK = N // BLK
NUM_BINS = 3
K0 = N // NUM_BINS
K1 = 2 * (N // NUM_BINS)
MAX_FINITE_BITS = 0x7F7FFFFF


def _fused_kernel(s2d_ref, s_row_ref, w_hbm, out_ref,
                  qmin_ref, qmax_ref, b0, b1, b2, sem0, sem1, sem2):
    j = pl.program_id(0)
    bufs = (b0, b1, b2)
    sems = (sem0, sem1, sem2)

    @pl.when(j == 0)
    def _start0():
        pltpu.make_async_copy(
            w_hbm.at[:, pl.ds(0, BLK)], b0, sem0).start()

    @pl.when(j == 0)
    def _bin():
        s2d = s2d_ref[:]

        def cnt_le(vbits):
            v = lax.bitcast_convert_type(vbits, jnp.float32)
            le = jnp.where(s2d <= v, 1.0, 0.0)
            return jnp.sum(le, axis=(0, 1), keepdims=True)

        def narrow(lo, hi, k_rank):
            ln = hi - lo
            step = jnp.maximum(jnp.right_shift(ln, 4), 1)
            idx = jnp.full((1, 1), 15, jnp.int32)
            for k in range(1, 16):
                b = cnt_le(lo + step * k) >= k_rank + 1
                idx = idx - jnp.where(b, 1, 0)
            nlo = lo + step * idx
            nhi = jnp.where(idx == 15, hi,
                            jnp.minimum(lo + step * (idx + 1), hi))
            return nlo, nhi

        def body(_, carry):
            lo0, hi0, lo1, hi1 = carry
            lo0, hi0 = narrow(lo0, hi0, K0)
            lo1, hi1 = narrow(lo1, hi1, K1)
            return lo0, hi0, lo1, hi1

        lo = jnp.full((1, 1), -1, jnp.int32)
        hi = jnp.full((1, 1), MAX_FINITE_BITS, jnp.int32)
        _, hi0, _, hi1 = lax.fori_loop(0, 10, body, (lo, hi, lo, hi))
        t0 = lax.bitcast_convert_type(hi0, jnp.float32)
        t1 = lax.bitcast_convert_type(hi1, jnp.float32)
        s_row = s_row_ref[:]
        half = jnp.where(s_row <= t0, 2.0, jnp.where(s_row <= t1, 8.0, 32.0))
        qmin_ref[:] = -half
        qmax_ref[:] = half - 1.0

    @pl.when(j == 0)
    def _start1():
        pltpu.make_async_copy(
            w_hbm.at[:, pl.ds(BLK, BLK)], b1, sem1).start()

    @pl.when(j + 2 < NBLK)
    def _prefetch():
        for par in (0, 1, 2):
            @pl.when((j + 2) % 3 == par)
            def _(par=par):
                pltpu.make_async_copy(
                    w_hbm.at[:, pl.ds((j + 2) * BLK, BLK)],
                    bufs[par], sems[par]).start()

    q_min = qmin_ref[:, pl.ds(j * BLK, BLK)]
    q_max = qmax_ref[:, pl.ds(j * BLK, BLK)]
    for par in (0, 1, 2):
        @pl.when(j % 3 == par)
        def _(par=par):
            pltpu.make_async_copy(
                w_hbm.at[:, pl.ds(j * BLK, BLK)],
                bufs[par], sems[par]).wait()
            w = bufs[par][:]
            mn = jnp.min(w, axis=0, keepdims=True)
            mx = jnp.max(w, axis=0, keepdims=True)
            scale = (mx - mn) / (q_max - q_min)
            scale = jnp.where(jnp.abs(scale) < 1e-6, jnp.float32(1e-6), scale)
            zp = jnp.clip(jnp.round(q_min - mn / scale), q_min, q_max)
            q = jnp.clip(jnp.round(w / scale), -128.0 - zp, 127.0 - zp)
            out_ref[:] = q * scale


def kernel(weight, scores):
    s_row = scores.reshape(1, N)
    s2d = scores.reshape(8, N // 8)
    out = pl.pallas_call(
        _fused_kernel,
        grid=(NBLK,),
        in_specs=[
            pl.BlockSpec((8, N // 8), lambda j: (0, 0)),
            pl.BlockSpec((1, N), lambda j: (0, 0)),
            pl.BlockSpec(memory_space=pl.ANY),
        ],
        out_specs=pl.BlockSpec((N, BLK), lambda j: (0, j)),
        out_shape=jax.ShapeDtypeStruct((N, N), jnp.float32),
        scratch_shapes=[
            pltpu.VMEM((1, N), jnp.float32),
            pltpu.VMEM((1, N), jnp.float32),
            pltpu.VMEM((N, BLK), jnp.float32),
            pltpu.VMEM((N, BLK), jnp.float32),
            pltpu.VMEM((N, BLK), jnp.float32),
            pltpu.SemaphoreType.DMA,
            pltpu.SemaphoreType.DMA,
            pltpu.SemaphoreType.DMA,
        ],
        compiler_params=pltpu.CompilerParams(
            dimension_semantics=("arbitrary",),
        ),
    )(s2d, s_row, weight)
    return out


# speedup vs baseline: 1.0731x; 1.0135x over previous
import jax
import jax.numpy as jnp
from jax import lax
from jax.experimental import pallas as pl
from jax.experimental.pallas import tpu as pltpu

N = 4096
BLK = 512
NBLK = N // BLK
NUM_BINS = 3
K0 = N // NUM_BINS
K1 = 2 * (N // NUM_BINS)
MAX_FINITE_BITS = 0x7F7FFFFF


def _fused_kernel(s2d_ref, s_row_ref, w_hbm, out_ref,
                  qmin_ref, qmax_ref, b0, b1, b2, b3,
                  sem0, sem1, sem2, sem3):
    j = pl.program_id(0)
    bufs = (b0, b1, b2, b3)
    sems = (sem0, sem1, sem2, sem3)

    @pl.when(j == 0)
    def _start0():
        pltpu.make_async_copy(
            w_hbm.at[:, pl.ds(0, BLK)], b0, sem0).start()

    @pl.when(j == 0)
    def _bin():
        s2d = s2d_ref[:]

        def cnt_le(vbits):
            v = lax.bitcast_convert_type(vbits, jnp.float32)
            le = jnp.where(s2d <= v, 1.0, 0.0)
            return jnp.sum(le, axis=(0, 1), keepdims=True)

        def narrow(lo, hi, k_rank):
            ln = hi - lo
            step = jnp.maximum(jnp.right_shift(ln, 4), 1)
            idx = jnp.full((1, 1), 15, jnp.int32)
            for k in range(1, 16):
                b = cnt_le(lo + step * k) >= k_rank + 1
                idx = idx - jnp.where(b, 1, 0)
            nlo = lo + step * idx
            nhi = jnp.where(idx == 15, hi,
                            jnp.minimum(lo + step * (idx + 1), hi))
            return nlo, nhi

        def body(_, carry):
            lo0, hi0, lo1, hi1 = carry
            lo0, hi0 = narrow(lo0, hi0, K0)
            lo1, hi1 = narrow(lo1, hi1, K1)
            return lo0, hi0, lo1, hi1

        lo = jnp.full((1, 1), -1, jnp.int32)
        hi = jnp.full((1, 1), MAX_FINITE_BITS, jnp.int32)
        _, hi0, _, hi1 = lax.fori_loop(0, 10, body, (lo, hi, lo, hi))
        t0 = lax.bitcast_convert_type(hi0, jnp.float32)
        t1 = lax.bitcast_convert_type(hi1, jnp.float32)
        s_row = s_row_ref[:]
        half = jnp.where(s_row <= t0, 2.0, jnp.where(s_row <= t1, 8.0, 32.0))
        qmin_ref[:] = -half
        qmax_ref[:] = half - 1.0

    @pl.when(j == 0)
    def _start1():
        pltpu.make_async_copy(
            w_hbm.at[:, pl.ds(BLK, BLK)], b1, sem1).start()

    @pl.when(j == 0)
    def _start2():
        pltpu.make_async_copy(
            w_hbm.at[:, pl.ds(2 * BLK, BLK)], b2, sem2).start()

    @pl.when(j + 3 < NBLK)
    def _prefetch():
        for par in (0, 1, 2, 3):
            @pl.when((j + 3) % 4 == par)
            def _(par=par):
                pltpu.make_async_copy(
                    w_hbm.at[:, pl.ds((j + 3) * BLK, BLK)],
                    bufs[par], sems[par]).start()

    q_min = qmin_ref[:, pl.ds(j * BLK, BLK)]
    q_max = qmax_ref[:, pl.ds(j * BLK, BLK)]
    for par in (0, 1, 2, 3):
        @pl.when(j % 4 == par)
        def _(par=par):
            pltpu.make_async_copy(
                w_hbm.at[:, pl.ds(j * BLK, BLK)],
                bufs[par], sems[par]).wait()
            w = bufs[par][:]
            mn = jnp.min(w, axis=0, keepdims=True)
            mx = jnp.max(w, axis=0, keepdims=True)
            scale = (mx - mn) / (q_max - q_min)
            scale = jnp.where(jnp.abs(scale) < 1e-6, jnp.float32(1e-6), scale)
            zp = jnp.clip(jnp.round(q_min - mn / scale), q_min, q_max)
            q = jnp.clip(jnp.round(w / scale), -128.0 - zp, 127.0 - zp)
            out_ref[:] = q * scale


def kernel(weight, scores):
    s_row = scores.reshape(1, N)
    s2d = scores.reshape(8, N // 8)
    out = pl.pallas_call(
        _fused_kernel,
        grid=(NBLK,),
        in_specs=[
            pl.BlockSpec((8, N // 8), lambda j: (0, 0)),
            pl.BlockSpec((1, N), lambda j: (0, 0)),
            pl.BlockSpec(memory_space=pl.ANY),
        ],
        out_specs=pl.BlockSpec((N, BLK), lambda j: (0, j)),
        out_shape=jax.ShapeDtypeStruct((N, N), jnp.float32),
        scratch_shapes=[
            pltpu.VMEM((1, N), jnp.float32),
            pltpu.VMEM((1, N), jnp.float32),
            pltpu.VMEM((N, BLK), jnp.float32),
            pltpu.VMEM((N, BLK), jnp.float32),
            pltpu.VMEM((N, BLK), jnp.float32),
            pltpu.VMEM((N, BLK), jnp.float32),
            pltpu.SemaphoreType.DMA,
            pltpu.SemaphoreType.DMA,
            pltpu.SemaphoreType.DMA,
            pltpu.SemaphoreType.DMA,
        ],
        compiler_params=pltpu.CompilerParams(
            dimension_semantics=("arbitrary",),
        ),
    )(s2d, s_row, weight)
    return out
